# Initial kernel scaffold; baseline (speedup 1.0000x reference)
import functools

import jax
import jax.numpy as jnp
from jax import lax
from jax.experimental import pallas as pl
from jax.experimental.pallas import tpu as pltpu
import jax.experimental.pallas.tpu_sc as plsc

N = 10000
E = 160000
D = 256
NC = 2
NS = 16
L = 16
NW = NC * NS
N_PAD = 10240
EW1 = 5008
E_PAD = NW * EW1
EW2 = E_PAD // NS
HALF = 5000
HPAD = 5120
RPT = HPAD // NS
CH = 64
PAD_DST = N + 16

_mesh = plsc.VectorSubcoreMesh(
    core_axis_name="c", subcore_axis_name="s", num_cores=NC, num_subcores=NS)


def _tc_body(x_ref, wfc_ref, wa_ref, z_ref, st_ref, b_ref, acc_ref):
    i = pl.program_id(0)
    z = jnp.dot(x_ref[...], wfc_ref[...], preferred_element_type=jnp.float32)
    z_ref[...] = z
    st = jnp.dot(z, wa_ref[...], preferred_element_type=jnp.float32)
    st_ref[...] = st
    ms = jnp.max(st[:, 0])
    mt = jnp.max(st[:, 1])

    @pl.when(i == 0)
    def _():
        acc_ref[0] = ms
        acc_ref[1] = mt

    @pl.when(i > 0)
    def _():
        acc_ref[0] = jnp.maximum(acc_ref[0], ms)
        acc_ref[1] = jnp.maximum(acc_ref[1], mt)

    @pl.when(i == pl.num_programs(0) - 1)
    def _():
        bb = acc_ref[0] + acc_ref[1]
        bb = jnp.where(bb >= 0.0, bb, 0.2 * bb)
        b_ref[...] = jnp.full((8, 128), bb, jnp.float32)


def _tc_stage(x, W_fc, Wa2):
    blk = 1000
    grid = N // blk
    return pl.pallas_call(
        _tc_body,
        grid=(grid,),
        in_specs=[
            pl.BlockSpec((blk, D), lambda i: (i, 0)),
            pl.BlockSpec((D, D), lambda i: (0, 0)),
            pl.BlockSpec((D, 128), lambda i: (0, 0)),
        ],
        out_specs=[
            pl.BlockSpec((blk, D), lambda i: (i, 0)),
            pl.BlockSpec((blk, 128), lambda i: (i, 0)),
            pl.BlockSpec((8, 128), lambda i: (0, 0)),
        ],
        out_shape=[
            jax.ShapeDtypeStruct((N, D), jnp.float32),
            jax.ShapeDtypeStruct((N, 128), jnp.float32),
            jax.ShapeDtypeStruct((8, 128), jnp.float32),
        ],
        scratch_shapes=[pltpu.SMEM((2,), jnp.float32)],
    )(x, W_fc, Wa2)


@functools.partial(
    pl.kernel,
    out_type=[
        jax.ShapeDtypeStruct((E_PAD,), jnp.float32),
        jax.ShapeDtypeStruct((NC, N_PAD), jnp.float32),
    ],
    mesh=_mesh,
    scratch_types=[
        pltpu.VMEM((EW1,), jnp.int32),
        pltpu.VMEM((EW1,), jnp.int32),
        pltpu.VMEM((N_PAD,), jnp.float32),
        pltpu.VMEM((N_PAD,), jnp.float32),
        pltpu.VMEM((N_PAD,), jnp.float32),
        pltpu.VMEM((EW1,), jnp.float32),
        pltpu.VMEM((L,), jnp.float32),
        pltpu.VMEM_SHARED((NS, N_PAD), jnp.float32),
        pltpu.VMEM((NS, N_PAD // NS), jnp.float32),
    ],
)
def _sc_edge(src_hbm, dst_hbm, s_hbm, t_hbm, b_hbm, ex_hbm, dpart_hbm,
             src_v, dst_v, s_v, t_v, den_v, ex_v, b_v, shared, red_v):
    c = lax.axis_index("c")
    sid = lax.axis_index("s")
    w = sid * NC + c
    base = w * EW1
    pltpu.sync_copy(src_hbm.at[pl.ds(base, EW1)], src_v)
    pltpu.sync_copy(dst_hbm.at[pl.ds(base, EW1)], dst_v)
    pltpu.sync_copy(s_hbm, s_v)
    pltpu.sync_copy(t_hbm, t_v)
    pltpu.sync_copy(b_hbm, b_v)
    bb = b_v[...]
    zv = jnp.zeros((L,), jnp.float32)

    def zero_body(i, carry):
        den_v[pl.ds(i * L, L)] = zv
        return carry

    lax.fori_loop(0, N_PAD // L, zero_body, 0)

    def win(i, carry):
        sl = pl.ds(i * L, L)
        isrc = src_v[sl]
        idst = dst_v[sl]
        sv = plsc.load_gather(s_v, [isrc])
        tv = plsc.load_gather(t_v, [idst])
        e = sv + tv
        e = jnp.where(e >= 0.0, e, 0.2 * e)
        exv = jnp.exp(e - bb)
        ex_v[sl] = exv
        plsc.addupdate_scatter(den_v, [idst], exv)
        return carry

    lax.fori_loop(0, EW1 // L, win, 0)
    pltpu.sync_copy(ex_v, ex_hbm.at[pl.ds(base, EW1)])

    pltpu.sync_copy(den_v, shared.at[sid])
    plsc.subcore_barrier()
    seg = N_PAD // NS
    for j in range(NS):
        pltpu.sync_copy(shared.at[j, pl.ds(sid * seg, seg)], red_v.at[j])

    def col(k, carry):
        csl = pl.ds(k * L, L)
        acc = red_v[0, csl]
        for j in range(1, NS):
            acc = acc + red_v[j, csl]
        den_v[csl] = acc
        return carry

    lax.fori_loop(0, seg // L, col, 0)
    pltpu.sync_copy(den_v.at[pl.ds(0, seg)], dpart_hbm.at[c, pl.ds(sid * seg, seg)])


@functools.partial(
    pl.kernel,
    out_type=jax.ShapeDtypeStruct((NC, HPAD, D), jnp.float32),
    mesh=_mesh,
    scratch_types=[
        pltpu.VMEM((EW2,), jnp.int32),
        pltpu.VMEM((EW2,), jnp.int32),
        pltpu.VMEM((EW2,), jnp.float32),
        pltpu.VMEM((EW2 + CH,), jnp.int32),
        pltpu.VMEM((EW2 + CH,), jnp.int32),
        pltpu.VMEM((EW2 + CH,), jnp.float32),
        pltpu.VMEM((CH, D), jnp.float32),
        pltpu.VMEM((CH,), jnp.int32),
        pltpu.VMEM((RPT,), jnp.float32),
        pltpu.VMEM((RPT,), jnp.float32),
        pltpu.VMEM_SHARED((HPAD, D), jnp.float32),
        pltpu.SemaphoreType.DMA,
    ],
)
def _sc_scatter(src_hbm, dst_hbm, ex_hbm, z_hbm, dpart_hbm, h_hbm,
                src_v, dst_v, ex_v, cs_v, cd_v, ce_v, rb, idx_st,
                dn0_v, dn1_v, h_sh, sem):
    c = lax.axis_index("c")
    sid = lax.axis_index("s")
    base = sid * EW2
    pltpu.sync_copy(src_hbm.at[pl.ds(base, EW2)], src_v)
    pltpu.sync_copy(dst_hbm.at[pl.ds(base, EW2)], dst_v)
    pltpu.sync_copy(ex_hbm.at[pl.ds(base, EW2)], ex_v)

    zv = jnp.zeros((L,), jnp.float32)

    def zrow(j, carry):
        for v in range(D // L):
            rb[j, pl.ds(v * L, L)] = zv
        return carry

    lax.fori_loop(0, CH, zrow, 0)
    r0 = sid * RPT
    for i in range(RPT // CH):
        pltpu.sync_copy(rb, h_sh.at[pl.ds(r0 + i * CH, CH)])
    plsc.subcore_barrier()

    lo = c * HALF
    hi = HALF + c * HPAD
    izero = jnp.zeros((L,), jnp.int32)

    def cwin(i, n):
        sl = pl.ds(i * L, L)
        d16 = dst_v[sl]
        m = (d16 >= lo) & (d16 < hi)
        cnt = plsc.all_reduce_population_count(m)
        plsc.store_compressed(cs_v.at[pl.ds(n, L)], src_v[sl], mask=m)
        plsc.store_compressed(cd_v.at[pl.ds(n, L)], d16 - lo, mask=m)
        plsc.store_compressed(ce_v.at[pl.ds(n, L)], ex_v[sl], mask=m)
        return n + cnt[0]

    n = lax.fori_loop(0, EW2 // L, cwin, 0)
    for j in range(CH // L):
        cs_v[pl.ds(n + j * L, L)] = izero
        cd_v[pl.ds(n + j * L, L)] = izero
        ce_v[pl.ds(n + j * L, L)] = zv
    nch = (n + CH - 1) // CH

    def chunk(k, carry):
        off = k * CH
        pltpu.make_async_copy(
            z_hbm.at[cs_v.at[pl.ds(off, CH)]], rb, sem).wait()
        pltpu.sync_copy(cd_v.at[pl.ds(off, CH)], idx_st)

        def scale(j, carry2):
            exj = ce_v[off + j]
            ev = jnp.full((L,), exj, jnp.float32)
            for v in range(D // L):
                sl2 = pl.ds(v * L, L)
                rb[j, sl2] = rb[j, sl2] * ev
            return carry2

        lax.fori_loop(0, CH, scale, 0)
        pltpu.sync_copy(rb, h_sh.at[idx_st], add=True)
        return carry

    lax.fori_loop(0, nch, chunk, 0)
    plsc.subcore_barrier()

    pltpu.sync_copy(dpart_hbm.at[0, pl.ds(c * HALF + r0, RPT)], dn0_v)
    pltpu.sync_copy(dpart_hbm.at[1, pl.ds(c * HALF + r0, RPT)], dn1_v)
    for i in range(RPT // CH):
        pltpu.sync_copy(h_sh.at[pl.ds(r0 + i * CH, CH)], rb)

        def fin(j, carry):
            d = dn0_v[i * CH + j] + dn1_v[i * CH + j]
            rinv = jnp.where(d > 0.0, 1.0 / d, 0.0)
            rv = jnp.full((L,), rinv, jnp.float32)
            for v in range(D // L):
                sl2 = pl.ds(v * L, L)
                rb[j, sl2] = rb[j, sl2] * rv
            return carry

        lax.fori_loop(0, CH, fin, 0)
        pltpu.sync_copy(rb, h_hbm.at[c, pl.ds(r0 + i * CH, CH)])


def kernel(x, edge_index, W_fc, W_attn):
    src = edge_index[0]
    dst = edge_index[1]
    pad = E_PAD - E
    src_p = jnp.concatenate([src, jnp.zeros((pad,), jnp.int32)])
    dst_p = jnp.concatenate([dst, jnp.full((pad,), PAD_DST, jnp.int32)])
    a = W_attn[:, 0]
    Wa2 = jnp.zeros((D, 128), jnp.float32).at[:, 0].set(a[:D]).at[:, 1].set(a[D:])
    z, st, b2d = _tc_stage(x, W_fc, Wa2)
    s_pad = jnp.zeros((N_PAD,), jnp.float32).at[:N].set(st[:, 0])
    t_pad = jnp.zeros((N_PAD,), jnp.float32).at[:N].set(st[:, 1])
    b_arr = jnp.full((L,), b2d[0, 0], jnp.float32)
    ex, dpart = _sc_edge(src_p, dst_p, s_pad, t_pad, b_arr)
    h2 = _sc_scatter(src_p, dst_p, ex, z, dpart)
    return jnp.concatenate([h2[0, :HALF], h2[1, :HALF]], axis=0)


# speedup vs baseline: 3.0765x; 3.0765x over previous
import functools

import jax
import jax.numpy as jnp
from jax import lax
from jax.experimental import pallas as pl
from jax.experimental.pallas import tpu as pltpu
import jax.experimental.pallas.tpu_sc as plsc

N = 10000
E = 160000
D = 256
NC = 2
NS = 16
L = 16
NW = NC * NS
N_PAD = 10240
EW1 = 5008
E_PAD = NW * EW1
PAD_DST = N + 16

RPW = N_PAD // NW
BLK = 5008
HCAP = 7168
CH2 = 32

_mesh = plsc.VectorSubcoreMesh(
    core_axis_name="c", subcore_axis_name="s", num_cores=NC, num_subcores=NS)
_sc_params = pltpu.CompilerParams(needs_layout_passes=False)


def _tc_body(x_ref, wfc_ref, wa_ref, z_ref, st_ref, b_ref, acc_ref):
    i = pl.program_id(0)
    z = jnp.dot(x_ref[...], wfc_ref[...], preferred_element_type=jnp.float32)
    z_ref[...] = z
    st = jnp.dot(z, wa_ref[...], preferred_element_type=jnp.float32)
    st_ref[...] = st
    ms = jnp.max(st[:, 0])
    mt = jnp.max(st[:, 1])

    @pl.when(i == 0)
    def _():
        acc_ref[0] = ms
        acc_ref[1] = mt

    @pl.when(i > 0)
    def _():
        acc_ref[0] = jnp.maximum(acc_ref[0], ms)
        acc_ref[1] = jnp.maximum(acc_ref[1], mt)

    @pl.when(i == pl.num_programs(0) - 1)
    def _():
        bb = acc_ref[0] + acc_ref[1]
        bb = jnp.where(bb >= 0.0, bb, 0.2 * bb)
        b_ref[...] = jnp.full((8, 128), bb, jnp.float32)


def _tc_stage(x, W_fc, Wa2):
    blk = 1000
    grid = N // blk
    return pl.pallas_call(
        _tc_body,
        grid=(grid,),
        in_specs=[
            pl.BlockSpec((blk, D), lambda i: (i, 0)),
            pl.BlockSpec((D, D), lambda i: (0, 0)),
            pl.BlockSpec((D, 128), lambda i: (0, 0)),
        ],
        out_specs=[
            pl.BlockSpec((blk, D), lambda i: (i, 0)),
            pl.BlockSpec((blk, 128), lambda i: (i, 0)),
            pl.BlockSpec((8, 128), lambda i: (0, 0)),
        ],
        out_shape=[
            jax.ShapeDtypeStruct((N, D), jnp.float32),
            jax.ShapeDtypeStruct((N, 128), jnp.float32),
            jax.ShapeDtypeStruct((8, 128), jnp.float32),
        ],
        scratch_shapes=[pltpu.SMEM((2,), jnp.float32)],
    )(x, W_fc, Wa2)


@functools.partial(
    pl.kernel,
    out_type=[
        jax.ShapeDtypeStruct((E_PAD,), jnp.float32),
        jax.ShapeDtypeStruct((NC * N_PAD,), jnp.float32),
    ],
    mesh=_mesh,
    scratch_types=[
        pltpu.VMEM((EW1,), jnp.int32),
        pltpu.VMEM((EW1,), jnp.int32),
        pltpu.VMEM((N_PAD,), jnp.float32),
        pltpu.VMEM((N_PAD,), jnp.float32),
        pltpu.VMEM((N_PAD,), jnp.float32),
        pltpu.VMEM((EW1,), jnp.float32),
        pltpu.VMEM((L,), jnp.float32),
        pltpu.VMEM_SHARED((NS, N_PAD), jnp.float32),
        pltpu.VMEM((NS, N_PAD // NS), jnp.float32),
    ],
    compiler_params=_sc_params,
)
def _sc_edge(src_hbm, dst_hbm, s_hbm, t_hbm, b_hbm, ex_hbm, dpart_hbm,
             src_v, dst_v, s_v, t_v, den_v, ex_v, b_v, shared, red_v):
    c = lax.axis_index("c")
    sid = lax.axis_index("s")
    w = sid * NC + c
    base = w * EW1
    pltpu.sync_copy(src_hbm.at[pl.ds(base, EW1)], src_v)
    pltpu.sync_copy(dst_hbm.at[pl.ds(base, EW1)], dst_v)
    pltpu.sync_copy(s_hbm, s_v)
    pltpu.sync_copy(t_hbm, t_v)
    pltpu.sync_copy(b_hbm, b_v)
    bb = b_v[...]
    zv = jnp.zeros((L,), jnp.float32)

    def zero_body(i, carry):
        den_v[pl.ds(i * L, L)] = zv
        return carry

    lax.fori_loop(0, N_PAD // L, zero_body, 0)

    def win(i, carry):
        sl = pl.ds(i * L, L)
        isrc = src_v[sl]
        idst = dst_v[sl]
        sv = plsc.load_gather(s_v, [isrc])
        tv = plsc.load_gather(t_v, [idst])
        e = sv + tv
        e = jnp.where(e >= 0.0, e, 0.2 * e)
        exv = jnp.exp(e - bb)
        ex_v[sl] = exv
        plsc.addupdate_scatter(den_v, [idst], exv)
        return carry

    lax.fori_loop(0, EW1 // L, win, 0)
    pltpu.sync_copy(ex_v, ex_hbm.at[pl.ds(base, EW1)])

    pltpu.sync_copy(den_v, shared.at[sid])
    plsc.subcore_barrier()
    seg = N_PAD // NS
    for j in range(NS):
        pltpu.sync_copy(shared.at[j, pl.ds(sid * seg, seg)], red_v.at[j])

    def col(k, carry):
        csl = pl.ds(k * L, L)
        acc = red_v[0, csl]
        for j in range(1, NS):
            acc = acc + red_v[j, csl]
        den_v[csl] = acc
        return carry

    lax.fori_loop(0, seg // L, col, 0)
    pltpu.sync_copy(den_v.at[pl.ds(0, seg)],
                    dpart_hbm.at[pl.ds(c * N_PAD + sid * seg, seg)])


@functools.partial(
    pl.kernel,
    out_type=jax.ShapeDtypeStruct((N_PAD * D,), jnp.float32),
    mesh=_mesh,
    scratch_types=[
        pltpu.VMEM((RPW * D,), jnp.float32),
        pltpu.VMEM((CH2 * D,), jnp.float32),
        pltpu.VMEM((BLK,), jnp.int32),
        pltpu.VMEM((BLK,), jnp.int32),
        pltpu.VMEM((BLK,), jnp.float32),
        pltpu.VMEM((HCAP,), jnp.int32),
        pltpu.VMEM((HCAP,), jnp.int32),
        pltpu.VMEM((HCAP,), jnp.float32),
        pltpu.VMEM((RPW,), jnp.float32),
        pltpu.VMEM((RPW,), jnp.float32),
        pltpu.SemaphoreType.DMA,
    ],
    compiler_params=_sc_params,
)
def _sc_scatter(src_hbm, dst_hbm, ex_hbm, z_hbm, dpart_hbm, h_hbm,
                h_loc, rb, src_v, dst_v, ex_v, cs_v, cd_v, ce_v,
                dn0_v, dn1_v, sem):
    c = lax.axis_index("c")
    sid = lax.axis_index("s")
    w = sid * NC + c
    lo = w * RPW
    hi = lo + RPW
    zv = jnp.zeros((L,), jnp.float32)
    izero = jnp.zeros((L,), jnp.int32)

    def zrow(j, carry):
        for v in range(D // L):
            h_loc[pl.ds(j * D + v * L, L)] = zv
        return carry

    lax.fori_loop(0, RPW, zrow, 0)

    def blk(bi, n):
        base = bi * BLK
        pltpu.sync_copy(src_hbm.at[pl.ds(base, BLK)], src_v)
        pltpu.sync_copy(dst_hbm.at[pl.ds(base, BLK)], dst_v)
        pltpu.sync_copy(ex_hbm.at[pl.ds(base, BLK)], ex_v)

        def cwin(i, n2):
            sl = pl.ds(i * L, L)
            d16 = dst_v[sl]
            m = (d16 >= lo) & (d16 < hi)
            cnt = plsc.all_reduce_population_count(m)
            plsc.store_compressed(cs_v.at[pl.ds(n2, L)], src_v[sl], mask=m)
            plsc.store_compressed(cd_v.at[pl.ds(n2, L)], d16 - lo, mask=m)
            plsc.store_compressed(ce_v.at[pl.ds(n2, L)], ex_v[sl], mask=m)
            return n2 + cnt[0]

        return lax.fori_loop(0, BLK // L, cwin, n)

    n = lax.fori_loop(0, NW, blk, 0)
    for j in range(CH2 // L):
        cs_v[pl.ds(n + j * L, L)] = izero
        cd_v[pl.ds(n + j * L, L)] = izero
        ce_v[pl.ds(n + j * L, L)] = zv
    nch = (n + CH2 - 1) // CH2

    def chunk(k, carry):
        off = k * CH2
        descs = []
        for g in range(CH2 // L):
            idx16 = cs_v[pl.ds(off + g * L, L)]
            for j2 in range(L):
                row = g * L + j2
                zoff = pl.multiple_of(idx16[j2] * D, D)
                dsc = pltpu.make_async_copy(
                    z_hbm.at[pl.ds(zoff, D)],
                    rb.at[pl.ds(row * D, D)], sem)
                dsc.start()
                descs.append(dsc)
        for dsc in descs:
            dsc.wait()

        def acc(g, carry2):
            gsl = pl.ds(off + g * L, L)
            ev16 = ce_v[gsl]
            dl16 = cd_v[gsl]
            for j2 in range(L):
                ev = jnp.full((L,), ev16[j2], jnp.float32)
                hoff = pl.multiple_of(dl16[j2] * D, D)
                row = g * L + j2
                for v in range(D // L):
                    hsl = pl.ds(hoff + v * L, L)
                    rsl = pl.ds(row * D + v * L, L)
                    h_loc[hsl] = h_loc[hsl] + rb[rsl] * ev
            return carry2

        lax.fori_loop(0, CH2 // L, acc, 0)
        return carry

    lax.fori_loop(0, nch, chunk, 0)

    pltpu.sync_copy(dpart_hbm.at[pl.ds(lo, RPW)], dn0_v)
    pltpu.sync_copy(dpart_hbm.at[pl.ds(N_PAD + lo, RPW)], dn1_v)

    def fin(g, carry):
        dsl = pl.ds(g * L, L)
        d16 = dn0_v[dsl] + dn1_v[dsl]
        rv16 = jnp.where(d16 > 0.0, 1.0 / d16, jnp.zeros((L,), jnp.float32))
        for j2 in range(L):
            rv = jnp.full((L,), rv16[j2], jnp.float32)
            row = g * L + j2
            for v in range(D // L):
                sl2 = pl.ds(row * D + v * L, L)
                h_loc[sl2] = h_loc[sl2] * rv
        return carry

    lax.fori_loop(0, RPW // L, fin, 0)
    pltpu.sync_copy(h_loc, h_hbm.at[pl.ds(lo * D, RPW * D)])


def kernel(x, edge_index, W_fc, W_attn):
    src = edge_index[0]
    dst = edge_index[1]
    pad = E_PAD - E
    src_p = jnp.concatenate([src, jnp.zeros((pad,), jnp.int32)])
    dst_p = jnp.concatenate([dst, jnp.full((pad,), PAD_DST, jnp.int32)])
    a = W_attn[:, 0]
    Wa2 = jnp.zeros((D, 128), jnp.float32).at[:, 0].set(a[:D]).at[:, 1].set(a[D:])
    z, st, b2d = _tc_stage(x, W_fc, Wa2)
    s_pad = jnp.zeros((N_PAD,), jnp.float32).at[:N].set(st[:, 0])
    t_pad = jnp.zeros((N_PAD,), jnp.float32).at[:N].set(st[:, 1])
    b_arr = jnp.full((L,), b2d[0, 0], jnp.float32)
    ex, dpart = _sc_edge(src_p, dst_p, s_pad, t_pad, b_arr)
    h_flat = _sc_scatter(src_p, dst_p, ex, z.reshape(N * D), dpart)
    return h_flat.reshape(N_PAD, D)[:N]


# speedup vs baseline: 3.5089x; 1.1406x over previous
import functools

import jax
import jax.numpy as jnp
from jax import lax
from jax.experimental import pallas as pl
from jax.experimental.pallas import tpu as pltpu
import jax.experimental.pallas.tpu_sc as plsc

N = 10000
E = 160000
D = 256
NC = 2
NS = 16
L = 16
NW = NC * NS
N_PAD = 10240
EW1 = 5008
E_PAD = NW * EW1
PAD_DST = N + 16

RPW = N_PAD // NW
BLK = 5008
HCAP = 7168
CH2 = 16

_mesh = plsc.VectorSubcoreMesh(
    core_axis_name="c", subcore_axis_name="s", num_cores=NC, num_subcores=NS)
_sc_params = pltpu.CompilerParams(needs_layout_passes=False)


def _tc_body(x_ref, wfc_ref, wa_ref, z_ref, st_ref, b_ref, acc_ref):
    i = pl.program_id(0)
    z = jnp.dot(x_ref[...], wfc_ref[...], preferred_element_type=jnp.float32)
    z_ref[...] = z
    st = jnp.dot(z, wa_ref[...], preferred_element_type=jnp.float32)
    st_ref[...] = st
    ms = jnp.max(st[:, 0])
    mt = jnp.max(st[:, 1])

    @pl.when(i == 0)
    def _():
        acc_ref[0] = ms
        acc_ref[1] = mt

    @pl.when(i > 0)
    def _():
        acc_ref[0] = jnp.maximum(acc_ref[0], ms)
        acc_ref[1] = jnp.maximum(acc_ref[1], mt)

    @pl.when(i == pl.num_programs(0) - 1)
    def _():
        bb = acc_ref[0] + acc_ref[1]
        bb = jnp.where(bb >= 0.0, bb, 0.2 * bb)
        b_ref[...] = jnp.full((8, 128), bb, jnp.float32)


def _tc_stage(x, W_fc, Wa2):
    blk = 1000
    grid = N // blk
    return pl.pallas_call(
        _tc_body,
        grid=(grid,),
        in_specs=[
            pl.BlockSpec((blk, D), lambda i: (i, 0)),
            pl.BlockSpec((D, D), lambda i: (0, 0)),
            pl.BlockSpec((D, 128), lambda i: (0, 0)),
        ],
        out_specs=[
            pl.BlockSpec((blk, D), lambda i: (i, 0)),
            pl.BlockSpec((blk, 128), lambda i: (i, 0)),
            pl.BlockSpec((8, 128), lambda i: (0, 0)),
        ],
        out_shape=[
            jax.ShapeDtypeStruct((N, D), jnp.float32),
            jax.ShapeDtypeStruct((N, 128), jnp.float32),
            jax.ShapeDtypeStruct((8, 128), jnp.float32),
        ],
        scratch_shapes=[pltpu.SMEM((2,), jnp.float32)],
    )(x, W_fc, Wa2)


@functools.partial(
    pl.kernel,
    out_type=[
        jax.ShapeDtypeStruct((E_PAD,), jnp.float32),
        jax.ShapeDtypeStruct((NC * N_PAD,), jnp.float32),
    ],
    mesh=_mesh,
    scratch_types=[
        pltpu.VMEM((EW1,), jnp.int32),
        pltpu.VMEM((EW1,), jnp.int32),
        pltpu.VMEM((N_PAD,), jnp.float32),
        pltpu.VMEM((N_PAD,), jnp.float32),
        pltpu.VMEM((N_PAD,), jnp.float32),
        pltpu.VMEM((EW1,), jnp.float32),
        pltpu.VMEM((L,), jnp.float32),
        pltpu.VMEM_SHARED((NS, N_PAD), jnp.float32),
        pltpu.VMEM((NS, N_PAD // NS), jnp.float32),
    ],
    compiler_params=_sc_params,
)
def _sc_edge(src_hbm, dst_hbm, s_hbm, t_hbm, b_hbm, ex_hbm, dpart_hbm,
             src_v, dst_v, s_v, t_v, den_v, ex_v, b_v, shared, red_v):
    c = lax.axis_index("c")
    sid = lax.axis_index("s")
    w = sid * NC + c
    base = w * EW1
    pltpu.sync_copy(src_hbm.at[pl.ds(base, EW1)], src_v)
    pltpu.sync_copy(dst_hbm.at[pl.ds(base, EW1)], dst_v)
    pltpu.sync_copy(s_hbm, s_v)
    pltpu.sync_copy(t_hbm, t_v)
    pltpu.sync_copy(b_hbm, b_v)
    bb = b_v[...]
    zv = jnp.zeros((L,), jnp.float32)

    def zero_body(i, carry):
        den_v[pl.ds(i * L, L)] = zv
        return carry

    lax.fori_loop(0, N_PAD // L, zero_body, 0)

    def win(i, carry):
        sl = pl.ds(i * L, L)
        isrc = src_v[sl]
        idst = dst_v[sl]
        sv = plsc.load_gather(s_v, [isrc])
        tv = plsc.load_gather(t_v, [idst])
        e = sv + tv
        e = jnp.where(e >= 0.0, e, 0.2 * e)
        exv = jnp.exp(e - bb)
        ex_v[sl] = exv
        plsc.addupdate_scatter(den_v, [idst], exv)
        return carry

    lax.fori_loop(0, EW1 // L, win, 0)
    pltpu.sync_copy(ex_v, ex_hbm.at[pl.ds(base, EW1)])

    pltpu.sync_copy(den_v, shared.at[sid])
    plsc.subcore_barrier()
    seg = N_PAD // NS
    for j in range(NS):
        pltpu.sync_copy(shared.at[j, pl.ds(sid * seg, seg)], red_v.at[j])

    def col(k, carry):
        csl = pl.ds(k * L, L)
        acc = red_v[0, csl]
        for j in range(1, NS):
            acc = acc + red_v[j, csl]
        den_v[csl] = acc
        return carry

    lax.fori_loop(0, seg // L, col, 0)
    pltpu.sync_copy(den_v.at[pl.ds(0, seg)],
                    dpart_hbm.at[pl.ds(c * N_PAD + sid * seg, seg)])


@functools.partial(
    pl.kernel,
    out_type=jax.ShapeDtypeStruct((N_PAD * D,), jnp.float32),
    mesh=_mesh,
    scratch_types=[
        pltpu.VMEM((RPW * D,), jnp.float32),
        pltpu.VMEM((2 * CH2 * D,), jnp.float32),
        pltpu.VMEM((BLK,), jnp.int32),
        pltpu.VMEM((BLK,), jnp.int32),
        pltpu.VMEM((BLK,), jnp.float32),
        pltpu.VMEM((HCAP,), jnp.int32),
        pltpu.VMEM((HCAP,), jnp.int32),
        pltpu.VMEM((HCAP,), jnp.float32),
        pltpu.VMEM((RPW,), jnp.float32),
        pltpu.VMEM((RPW,), jnp.float32),
        pltpu.SemaphoreType.DMA,
        pltpu.SemaphoreType.DMA,
    ],
    compiler_params=_sc_params,
)
def _sc_scatter(src_hbm, dst_hbm, ex_hbm, z_hbm, dpart_hbm, h_hbm,
                h_loc, rb, src_v, dst_v, ex_v, cs_v, cd_v, ce_v,
                dn0_v, dn1_v, sem0, sem1):
    c = lax.axis_index("c")
    sid = lax.axis_index("s")
    w = sid * NC + c
    lo = w * RPW
    hi = lo + RPW
    zv = jnp.zeros((L,), jnp.float32)
    izero = jnp.zeros((L,), jnp.int32)

    def zrow(j, carry):
        for v in range(D // L):
            h_loc[pl.ds(j * D + v * L, L)] = zv
        return carry

    lax.fori_loop(0, RPW, zrow, 0)

    def blk(bi, n):
        base = bi * BLK
        pltpu.sync_copy(src_hbm.at[pl.ds(base, BLK)], src_v)
        pltpu.sync_copy(dst_hbm.at[pl.ds(base, BLK)], dst_v)
        pltpu.sync_copy(ex_hbm.at[pl.ds(base, BLK)], ex_v)

        def cwin(i, n2):
            sl = pl.ds(i * L, L)
            d16 = dst_v[sl]
            m = (d16 >= lo) & (d16 < hi)
            cnt = plsc.all_reduce_population_count(m)
            plsc.store_compressed(cs_v.at[pl.ds(n2, L)], src_v[sl], mask=m)
            plsc.store_compressed(cd_v.at[pl.ds(n2, L)], d16 - lo, mask=m)
            plsc.store_compressed(ce_v.at[pl.ds(n2, L)], ex_v[sl], mask=m)
            return n2 + cnt[0]

        return lax.fori_loop(0, BLK // L, cwin, n)

    n = lax.fori_loop(0, NW, blk, 0)
    for j in range(CH2 // L):
        cs_v[pl.ds(n + j * L, L)] = izero
        cd_v[pl.ds(n + j * L, L)] = izero
        ce_v[pl.ds(n + j * L, L)] = zv
    nch = (n + CH2 - 1) // CH2

    def fire(k, buf, sem):
        off = k * CH2
        idx16 = cs_v[pl.ds(off, L)]
        for j2 in range(L):
            zoff = pl.multiple_of(idx16[j2] * D, D)
            pltpu.make_async_copy(
                z_hbm.at[pl.ds(zoff, D)],
                rb.at[pl.ds(buf * (CH2 * D) + j2 * D, D)], sem).start()

    def drain_rmw(k, buf, sem):
        pltpu.make_async_copy(
            z_hbm.at[pl.ds(0, CH2 * D)],
            rb.at[pl.ds(buf * (CH2 * D), CH2 * D)], sem).wait()
        gsl = pl.ds(k * CH2, L)
        ev16 = ce_v[gsl]
        dl16 = cd_v[gsl]
        for j2 in range(L):
            ev = jnp.full((L,), ev16[j2], jnp.float32)
            hoff = pl.multiple_of(dl16[j2] * D, D)
            for v in range(D // L):
                hsl = pl.ds(hoff + v * L, L)
                rsl = pl.ds(buf * (CH2 * D) + j2 * D + v * L, L)
                h_loc[hsl] = h_loc[hsl] + rb[rsl] * ev

    fire(0, 0, sem0)

    def chunk2(kk, carry):
        k0 = kk * 2
        k1 = k0 + 1

        @pl.when(k1 < nch)
        def _():
            fire(k1, 1, sem1)

        drain_rmw(k0, 0, sem0)

        @pl.when(k1 < nch)
        def _():
            @pl.when(k1 + 1 < nch)
            def _():
                fire(k1 + 1, 0, sem0)

            drain_rmw(k1, 1, sem1)

        return carry

    lax.fori_loop(0, (nch + 1) // 2, chunk2, 0)

    pltpu.sync_copy(dpart_hbm.at[pl.ds(lo, RPW)], dn0_v)
    pltpu.sync_copy(dpart_hbm.at[pl.ds(N_PAD + lo, RPW)], dn1_v)

    def fin(g, carry):
        dsl = pl.ds(g * L, L)
        d16 = dn0_v[dsl] + dn1_v[dsl]
        rv16 = jnp.where(d16 > 0.0, 1.0 / d16, jnp.zeros((L,), jnp.float32))
        for j2 in range(L):
            rv = jnp.full((L,), rv16[j2], jnp.float32)
            row = g * L + j2
            for v in range(D // L):
                sl2 = pl.ds(row * D + v * L, L)
                h_loc[sl2] = h_loc[sl2] * rv
        return carry

    lax.fori_loop(0, RPW // L, fin, 0)
    pltpu.sync_copy(h_loc, h_hbm.at[pl.ds(lo * D, RPW * D)])


def kernel(x, edge_index, W_fc, W_attn):
    src = edge_index[0]
    dst = edge_index[1]
    pad = E_PAD - E
    src_p = jnp.concatenate([src, jnp.zeros((pad,), jnp.int32)])
    dst_p = jnp.concatenate([dst, jnp.full((pad,), PAD_DST, jnp.int32)])
    a = W_attn[:, 0]
    Wa2 = jnp.zeros((D, 128), jnp.float32).at[:, 0].set(a[:D]).at[:, 1].set(a[D:])
    z, st, b2d = _tc_stage(x, W_fc, Wa2)
    s_pad = jnp.zeros((N_PAD,), jnp.float32).at[:N].set(st[:, 0])
    t_pad = jnp.zeros((N_PAD,), jnp.float32).at[:N].set(st[:, 1])
    b_arr = jnp.full((L,), b2d[0, 0], jnp.float32)
    ex, dpart = _sc_edge(src_p, dst_p, s_pad, t_pad, b_arr)
    h_flat = _sc_scatter(src_p, dst_p, ex, z.reshape(N * D), dpart)
    return h_flat.reshape(N_PAD, D)[:N]
